# Initial kernel scaffold; baseline (speedup 1.0000x reference)
import functools

import jax
import jax.numpy as jnp
from jax import lax
from jax.experimental import pallas as pl
from jax.experimental.pallas import tpu as pltpu
from jax.experimental.pallas import tpu_sc as plsc

N_EDGES = 320000
D_FEAT = 128
N_SEGMENTS = 10000

NUM_CORES = 2
NUM_SUBCORES = 16
NUM_TILES = NUM_CORES * NUM_SUBCORES
ROWS_PER_TILE = N_EDGES // NUM_TILES
CHUNK = 80
CHUNKS_PER_TILE = ROWS_PER_TILE // CHUNK
SEG_PAD = 10240
SEG_PER_TILE = SEG_PAD // NUM_SUBCORES
ZCHUNK = 80


def _sc_partial_sums(x, idx2d):
    mesh = plsc.VectorSubcoreMesh(core_axis_name="c", subcore_axis_name="s")

    @functools.partial(
        pl.kernel,
        out_type=jax.ShapeDtypeStruct((NUM_CORES, SEG_PAD, D_FEAT), jnp.float32),
        mesh=mesh,
        scratch_types=[
            pltpu.VMEM((CHUNKS_PER_TILE, CHUNK), jnp.int32),
            pltpu.VMEM((CHUNK, D_FEAT), jnp.float32),
            pltpu.VMEM((ZCHUNK, D_FEAT), jnp.float32),
            pltpu.VMEM_SHARED((SEG_PAD, D_FEAT), jnp.float32),
        ],
    )
    def k(x_hbm, idx_hbm, out_hbm, idx_v, buf, zbuf, acc):
        c = lax.axis_index("c")
        s = lax.axis_index("s")

        @pl.loop(0, ZCHUNK)
        def _zrow(i):
            @pl.loop(0, D_FEAT, step=16)
            def _zlane(j):
                zbuf[i, pl.ds(j, 16)] = jnp.zeros((16,), jnp.float32)

        seg_base = s * SEG_PER_TILE

        @pl.loop(0, SEG_PER_TILE, step=ZCHUNK)
        def _clear(r):
            pltpu.sync_copy(zbuf, acc.at[pl.ds(seg_base + r, ZCHUNK)])

        plsc.subcore_barrier()

        tile = s * NUM_CORES + c
        row0 = tile * ROWS_PER_TILE
        pltpu.sync_copy(
            idx_hbm.at[pl.ds(tile * CHUNKS_PER_TILE, CHUNKS_PER_TILE)], idx_v
        )

        @pl.loop(0, CHUNKS_PER_TILE)
        def _chunk(g):
            pltpu.sync_copy(x_hbm.at[pl.ds(row0 + g * CHUNK, CHUNK)], buf)
            pltpu.sync_copy(buf, acc.at[idx_v.at[g]], add=True)

        plsc.subcore_barrier()

        pltpu.sync_copy(
            acc.at[pl.ds(seg_base, SEG_PER_TILE)],
            out_hbm.at[c].at[pl.ds(seg_base, SEG_PER_TILE)],
        )

    return k(x, idx2d)


def _combine(partials):
    def body(p_ref, o_ref):
        o_ref[...] = p_ref[0] + p_ref[1]

    blk = 1250

    return pl.pallas_call(
        body,
        grid=(N_SEGMENTS // blk,),
        in_specs=[
            pl.BlockSpec((NUM_CORES, blk, D_FEAT), lambda i: (0, i, 0)),
        ],
        out_specs=pl.BlockSpec((blk, D_FEAT), lambda i: (i, 0)),
        out_shape=jax.ShapeDtypeStruct((N_SEGMENTS, D_FEAT), jnp.float32),
    )(partials)


def kernel(x, idx):
    idx2d = idx.astype(jnp.int32).reshape(N_EDGES // CHUNK, CHUNK)
    partials = _sc_partial_sums(x, idx2d)
    return _combine(partials[:, :N_SEGMENTS, :])


# speedup vs baseline: 4.3571x; 4.3571x over previous
import functools

import jax
import jax.numpy as jnp
from jax import lax
from jax.experimental import pallas as pl
from jax.experimental.pallas import tpu as pltpu
from jax.experimental.pallas import tpu_sc as plsc

N_EDGES = 320000
D_FEAT = 128
N_SEGMENTS = 10000

NUM_CORES = 2
NUM_SUBCORES = 16
NUM_TILES = NUM_CORES * NUM_SUBCORES
CHUNK = 128
NCHUNKS = N_EDGES // CHUNK
BASE_CHUNKS = NCHUNKS // NUM_TILES
EXTRA_TILES = NCHUNKS % NUM_TILES
SEG_PAD = 10240
SEG_PER_TILE = SEG_PAD // NUM_SUBCORES
ZCHUNK = 80


def _sc_partial_sums(x3, idx3):
    mesh = plsc.VectorSubcoreMesh(core_axis_name="c", subcore_axis_name="s")

    @functools.partial(
        pl.kernel,
        out_type=jax.ShapeDtypeStruct((NUM_CORES, SEG_PAD, D_FEAT), jnp.float32),
        mesh=mesh,
        scratch_types=[
            pltpu.VMEM((1, CHUNK), jnp.int32),
            pltpu.VMEM((CHUNK, D_FEAT), jnp.float32),
            pltpu.VMEM((ZCHUNK, D_FEAT), jnp.float32),
            pltpu.VMEM_SHARED((SEG_PAD, D_FEAT), jnp.float32),
        ],
    )
    def k(x_hbm, idx_hbm, out_hbm, idx_v, buf, zbuf, acc):
        c = lax.axis_index("c")
        s = lax.axis_index("s")
        tile = s * NUM_CORES + c

        @pl.loop(0, ZCHUNK)
        def _zrow(i):
            @pl.loop(0, D_FEAT, step=16)
            def _zlane(j):
                zbuf[i, pl.ds(j, 16)] = jnp.zeros((16,), jnp.float32)

        seg_base = pl.multiple_of(s * SEG_PER_TILE, 8)

        @pl.loop(0, SEG_PER_TILE, step=ZCHUNK)
        def _clear(r):
            pltpu.sync_copy(zbuf, acc.at[pl.ds(seg_base + r, ZCHUNK)])

        plsc.subcore_barrier()

        start = BASE_CHUNKS * tile + jnp.minimum(tile, EXTRA_TILES)
        nchunks = BASE_CHUNKS + jnp.where(tile < EXTRA_TILES, 1, 0)

        def do_chunk(j):
            pltpu.sync_copy(idx_hbm.at[j], idx_v)
            pltpu.sync_copy(x_hbm.at[j], buf)
            pltpu.sync_copy(buf, acc.at[idx_v.at[0]], add=True)

        @pl.loop(0, BASE_CHUNKS)
        def _chunk(g):
            do_chunk(start + g)

        @pl.when(nchunks > BASE_CHUNKS)
        def _tail():
            do_chunk(start + BASE_CHUNKS)

        plsc.subcore_barrier()

        pltpu.sync_copy(
            acc.at[pl.ds(seg_base, SEG_PER_TILE)],
            out_hbm.at[c].at[pl.ds(seg_base, SEG_PER_TILE)],
        )

    return k(x3, idx3)


def _combine(partials):
    def body(p_ref, o_ref):
        o_ref[...] = p_ref[0] + p_ref[1]

    blk = 1000

    return pl.pallas_call(
        body,
        grid=(N_SEGMENTS // blk,),
        in_specs=[
            pl.BlockSpec((NUM_CORES, blk, D_FEAT), lambda i: (0, i, 0)),
        ],
        out_specs=pl.BlockSpec((blk, D_FEAT), lambda i: (i, 0)),
        out_shape=jax.ShapeDtypeStruct((N_SEGMENTS, D_FEAT), jnp.float32),
    )(partials)


def kernel(x, idx):
    x3 = x.reshape(NCHUNKS, CHUNK, D_FEAT)
    idx3 = idx.astype(jnp.int32).reshape(NCHUNKS, 1, CHUNK)
    partials = _sc_partial_sums(x3, idx3)
    return _combine(partials[:, :N_SEGMENTS, :])


# speedup vs baseline: 7.1991x; 1.6523x over previous
import functools

import jax
import jax.numpy as jnp
from jax import lax
from jax.experimental import pallas as pl
from jax.experimental.pallas import tpu as pltpu
from jax.experimental.pallas import tpu_sc as plsc

N_EDGES = 320000
D_FEAT = 128
N_SEGMENTS = 10000

NUM_CORES = 2
NUM_SUBCORES = 16
NUM_TILES = NUM_CORES * NUM_SUBCORES
CHUNK = 128
NCHUNKS = N_EDGES // CHUNK
BASE_CHUNKS = NCHUNKS // NUM_TILES
EXTRA_TILES = NCHUNKS % NUM_TILES
SEG_PAD = 10240
SEG_PER_TILE = SEG_PAD // NUM_SUBCORES
ZCHUNK = 80


def _sc_partial_sums(x3, idx3):
    mesh = plsc.VectorSubcoreMesh(core_axis_name="c", subcore_axis_name="s")

    @functools.partial(
        pl.kernel,
        out_type=jax.ShapeDtypeStruct((NUM_CORES, SEG_PAD, D_FEAT), jnp.float32),
        mesh=mesh,
        scratch_types=[
            pltpu.VMEM((BASE_CHUNKS, 1, CHUNK), jnp.int32),
            pltpu.VMEM((1, CHUNK), jnp.int32),
            pltpu.VMEM((CHUNK, D_FEAT), jnp.float32),
            pltpu.VMEM((CHUNK, D_FEAT), jnp.float32),
            pltpu.VMEM_SHARED((SEG_PAD, D_FEAT), jnp.float32),
            pltpu.SemaphoreType.DMA,
            pltpu.SemaphoreType.DMA,
        ],
    )
    def k(x_hbm, idx_hbm, out_hbm, idx_v, idx_tail, buf0, buf1, acc,
          sem0, sem1):
        c = lax.axis_index("c")
        s = lax.axis_index("s")
        tile = s * NUM_CORES + c

        @pl.loop(0, CHUNK)
        def _zrow(i):
            @pl.loop(0, D_FEAT, step=16)
            def _zlane(j):
                buf0[i, pl.ds(j, 16)] = jnp.zeros((16,), jnp.float32)

        seg_base = pl.multiple_of(s * SEG_PER_TILE, 8)

        @pl.loop(0, SEG_PER_TILE, step=CHUNK)
        def _clear(r):
            pltpu.sync_copy(buf0, acc.at[pl.ds(seg_base + r, CHUNK)])

        plsc.subcore_barrier()

        start = BASE_CHUNKS * tile + jnp.minimum(tile, EXTRA_TILES)

        pltpu.sync_copy(idx_hbm.at[pl.ds(start, BASE_CHUNKS)], idx_v)

        pltpu.make_async_copy(x_hbm.at[start], buf0, sem0).start()
        pltpu.make_async_copy(x_hbm.at[start + 1], buf1, sem1).start()

        @pl.loop(0, BASE_CHUNKS, step=2)
        def _chunk(g):
            for b, buf, sem in ((0, buf0, sem0), (1, buf1, sem1)):
                pltpu.make_async_copy(x_hbm.at[start + g + b], buf, sem).wait()
                pltpu.sync_copy(buf, acc.at[idx_v.at[g + b].at[0]], add=True)

                @pl.when(g + b + 2 < BASE_CHUNKS)
                def _prefetch(buf=buf, sem=sem, off=b + 2):
                    pltpu.make_async_copy(
                        x_hbm.at[start + g + off], buf, sem
                    ).start()

        @pl.when(tile < EXTRA_TILES)
        def _tail():
            j = start + BASE_CHUNKS
            pltpu.sync_copy(idx_hbm.at[j], idx_tail)
            pltpu.sync_copy(x_hbm.at[j], buf0)
            pltpu.sync_copy(buf0, acc.at[idx_tail.at[0]], add=True)

        plsc.subcore_barrier()

        pltpu.sync_copy(
            acc.at[pl.ds(seg_base, SEG_PER_TILE)],
            out_hbm.at[c].at[pl.ds(seg_base, SEG_PER_TILE)],
        )

    return k(x3, idx3)


def _combine(partials):
    def body(p_ref, o_ref):
        o_ref[...] = p_ref[0] + p_ref[1]

    blk = 1000

    return pl.pallas_call(
        body,
        grid=(N_SEGMENTS // blk,),
        in_specs=[
            pl.BlockSpec((NUM_CORES, blk, D_FEAT), lambda i: (0, i, 0)),
        ],
        out_specs=pl.BlockSpec((blk, D_FEAT), lambda i: (i, 0)),
        out_shape=jax.ShapeDtypeStruct((N_SEGMENTS, D_FEAT), jnp.float32),
    )(partials)


def kernel(x, idx):
    x3 = x.reshape(NCHUNKS, CHUNK, D_FEAT)
    idx3 = idx.astype(jnp.int32).reshape(NCHUNKS, 1, CHUNK)
    partials = _sc_partial_sums(x3, idx3)
    return _combine(partials[:, :N_SEGMENTS, :])


# speedup vs baseline: 7.5188x; 1.0444x over previous
import functools

import jax
import jax.numpy as jnp
from jax import lax
from jax.experimental import pallas as pl
from jax.experimental.pallas import tpu as pltpu
from jax.experimental.pallas import tpu_sc as plsc

N_EDGES = 320000
D_FEAT = 128
N_SEGMENTS = 10000

NUM_CORES = 2
NUM_SUBCORES = 16
NUM_TILES = NUM_CORES * NUM_SUBCORES
CHUNK = 128
NCHUNKS = N_EDGES // CHUNK
BASE_CHUNKS = NCHUNKS // NUM_TILES
EXTRA_TILES = NCHUNKS % NUM_TILES
SEG_PAD = 10240
SEG_PER_TILE = SEG_PAD // NUM_SUBCORES
ZCHUNK = 80


def _sc_partial_sums(x3, idx3):
    mesh = plsc.VectorSubcoreMesh(core_axis_name="c", subcore_axis_name="s")

    @functools.partial(
        pl.kernel,
        out_type=jax.ShapeDtypeStruct((NUM_CORES, SEG_PAD, D_FEAT), jnp.float32),
        mesh=mesh,
        scratch_types=[
            pltpu.VMEM((BASE_CHUNKS, 1, CHUNK), jnp.int32),
            pltpu.VMEM((1, CHUNK), jnp.int32),
            pltpu.VMEM((CHUNK, D_FEAT), jnp.float32),
            pltpu.VMEM((CHUNK, D_FEAT), jnp.float32),
            pltpu.VMEM_SHARED((SEG_PAD, D_FEAT), jnp.float32),
            pltpu.SemaphoreType.DMA,
            pltpu.SemaphoreType.DMA,
        ],
    )
    def k(x_hbm, idx_hbm, out_hbm, idx_v, idx_tail, buf0, buf1, acc,
          sem0, sem1):
        c = lax.axis_index("c")
        s = lax.axis_index("s")
        tile = s * NUM_CORES + c

        @pl.loop(0, CHUNK)
        def _zrow(i):
            @pl.loop(0, D_FEAT, step=16)
            def _zlane(j):
                buf0[i, pl.ds(j, 16)] = jnp.zeros((16,), jnp.float32)

        seg_base = pl.multiple_of(s * SEG_PER_TILE, 8)

        @pl.loop(0, SEG_PER_TILE, step=CHUNK)
        def _clear(r):
            pltpu.sync_copy(buf0, acc.at[pl.ds(seg_base + r, CHUNK)])

        plsc.subcore_barrier()

        start = BASE_CHUNKS * tile + jnp.minimum(tile, EXTRA_TILES)

        pltpu.sync_copy(idx_hbm.at[pl.ds(start, BASE_CHUNKS)], idx_v)

        pltpu.make_async_copy(x_hbm.at[start], buf0, sem0).start()
        pltpu.make_async_copy(x_hbm.at[start + 1], buf1, sem1).start()

        @pl.loop(0, BASE_CHUNKS, step=2)
        def _chunk(g):
            for b, buf, sem in ((0, buf0, sem0), (1, buf1, sem1)):
                pltpu.make_async_copy(x_hbm.at[start + g + b], buf, sem).wait()
                pltpu.sync_copy(buf, acc.at[idx_v.at[g + b].at[0]], add=True)

                @pl.when(g + b + 2 < BASE_CHUNKS)
                def _prefetch(buf=buf, sem=sem, off=b + 2):
                    pltpu.make_async_copy(
                        x_hbm.at[start + g + off], buf, sem
                    ).start()

        @pl.when(tile < EXTRA_TILES)
        def _tail():
            j = start + BASE_CHUNKS
            pltpu.sync_copy(idx_hbm.at[j], idx_tail)
            pltpu.sync_copy(x_hbm.at[j], buf0)
            pltpu.sync_copy(buf0, acc.at[idx_tail.at[0]], add=True)

        plsc.subcore_barrier()

        pltpu.sync_copy(
            acc.at[pl.ds(seg_base, SEG_PER_TILE)],
            out_hbm.at[c].at[pl.ds(seg_base, SEG_PER_TILE)],
        )

    return k(x3, idx3)


def _combine(partials):
    def body(p_ref, o_ref):
        o_ref[...] = p_ref[0] + p_ref[1]

    blk = 1000

    return pl.pallas_call(
        body,
        grid=(N_SEGMENTS // blk,),
        in_specs=[
            pl.BlockSpec((NUM_CORES, blk, D_FEAT), lambda i: (0, i, 0)),
        ],
        out_specs=pl.BlockSpec((blk, D_FEAT), lambda i: (i, 0)),
        out_shape=jax.ShapeDtypeStruct((N_SEGMENTS, D_FEAT), jnp.float32),
    )(partials)


def kernel(x, idx):
    x3 = x.reshape(NCHUNKS, CHUNK, D_FEAT)
    idx3 = idx.astype(jnp.int32).reshape(NCHUNKS, 1, CHUNK)
    partials = _sc_partial_sums(x3, idx3)
    return _combine(partials)
